# Initial kernel scaffold; baseline (speedup 1.0000x reference)
import functools

import jax
import jax.numpy as jnp
from jax import lax
from jax.experimental import pallas as pl
from jax.experimental.pallas import tpu as pltpu
from jax.experimental.pallas import tpu_sc as plsc

MAX_POSITION = 8192
EMBED_DIM = 1024
BATCH = 4
SEQ_LEN = 8192

_INFO = plsc.get_sparse_core_info()
_NC = _INFO.num_cores
_NS = _INFO.num_subcores
_NW = _NC * _NS

_B = BATCH * SEQ_LEN
_B_PER_W = _B // _NW
_CHUNK = 32
_N_CHUNKS = _B_PER_W // _CHUNK


def _gather_body(idx_hbm, table_hbm, out_hbm, idx_v, rows0, rows1, g0, g1, s0, s1):
    wid = lax.axis_index("s") * _NC + lax.axis_index("c")
    base = wid * _B_PER_W
    pltpu.sync_copy(idx_hbm.at[pl.ds(base, _B_PER_W)], idx_v)

    rows = (rows0, rows1)
    gsem = (g0, g1)
    ssem = (s0, s1)

    def gather_copy(i, buf):
        off = pl.multiple_of(i * _CHUNK, _CHUNK)
        return pltpu.make_async_copy(
            table_hbm.at[idx_v.at[pl.ds(off, _CHUNK)]], rows[buf], gsem[buf]
        )

    def store_copy(i, buf):
        off = pl.multiple_of(i * _CHUNK, _CHUNK)
        return pltpu.make_async_copy(
            rows[buf], out_hbm.at[pl.ds(base + off, _CHUNK)], ssem[buf]
        )

    gather_copy(0, 0).start()

    def step(i, _):
        nxt = i + 1

        @pl.when(nxt < _N_CHUNKS)
        def _():
            @pl.when(i >= 1)
            def _():
                store_copy(i - 1, nxt % 2).wait()

            gather_copy(nxt, nxt % 2).start()

        gather_copy(i, i % 2).wait()
        store_copy(i, i % 2).start()
        return 0

    lax.fori_loop(0, _N_CHUNKS, step, 0)
    store_copy(_N_CHUNKS - 2, _N_CHUNKS % 2).wait()
    store_copy(_N_CHUNKS - 1, (_N_CHUNKS - 1) % 2).wait()


@jax.jit
def _embed_gather(position_ids_flat, table):
    mesh = plsc.VectorSubcoreMesh(core_axis_name="c", subcore_axis_name="s")
    kern = functools.partial(
        pl.kernel,
        mesh=mesh,
        out_type=jax.ShapeDtypeStruct((_B, EMBED_DIM), jnp.float32),
        scratch_types=[
            pltpu.VMEM((_B_PER_W,), jnp.int32),
            pltpu.VMEM((_CHUNK, EMBED_DIM), jnp.float32),
            pltpu.VMEM((_CHUNK, EMBED_DIM), jnp.float32),
            pltpu.SemaphoreType.DMA,
            pltpu.SemaphoreType.DMA,
            pltpu.SemaphoreType.DMA,
            pltpu.SemaphoreType.DMA,
        ],
    )(_gather_body)
    return kern(position_ids_flat, table)


def kernel(position_ids, table):
    flat = position_ids.reshape(-1).astype(jnp.int32)
    out = _embed_gather(flat, table)
    return out.reshape(BATCH, SEQ_LEN, EMBED_DIM)


# speedup vs baseline: 2.2596x; 2.2596x over previous
import functools

import jax
import jax.numpy as jnp
from jax import lax
from jax.experimental import pallas as pl
from jax.experimental.pallas import tpu as pltpu
from jax.experimental.pallas import tpu_sc as plsc

MAX_POSITION = 8192
EMBED_DIM = 1024
BATCH = 4
SEQ_LEN = 8192

_INFO = plsc.get_sparse_core_info()
_NC = _INFO.num_cores
_NS = _INFO.num_subcores
_NW = _NC * _NS

_B = BATCH * SEQ_LEN
_B_PER_W = _B // _NW
_CHUNK = 32
_NBUF = 2
_N_CHUNKS = _B_PER_W // _CHUNK


def _gather_body(idx_hbm, table_hbm, out_hbm, idx_v, rows0, rows1, g0, g1, s0, s1):
    wid = lax.axis_index("s") * _NC + lax.axis_index("c")
    base = wid * _B_PER_W
    pltpu.sync_copy(idx_hbm.at[pl.ds(base, _B_PER_W)], idx_v)

    rows = (rows0, rows1)
    gsem = (g0, g1)
    ssem = (s0, s1)

    def gather_copy(i, buf):
        off = pl.multiple_of(i * _CHUNK, _CHUNK)
        return pltpu.make_async_copy(
            table_hbm.at[idx_v.at[pl.ds(off, _CHUNK)]], rows[buf], gsem[buf]
        )

    def store_copy(i, buf):
        off = pl.multiple_of(i * _CHUNK, _CHUNK)
        return pltpu.make_async_copy(
            rows[buf], out_hbm.at[pl.ds(base + off, _CHUNK)], ssem[buf]
        )

    for b in range(_NBUF):
        gather_copy(b, b).start()

    def group(g, _):
        cb = g * _NBUF
        for b in range(_NBUF):
            gather_copy(cb + b, b).wait()
            store_copy(cb + b, b).start()
        for b in range(_NBUF):
            @pl.when(cb + _NBUF + b < _N_CHUNKS)
            def _(b=b, cb=cb):
                store_copy(cb + b, b).wait()
                gather_copy(cb + _NBUF + b, b).start()
        return 0

    lax.fori_loop(0, _N_CHUNKS // _NBUF, group, 0)
    for b in range(_NBUF):
        store_copy(_N_CHUNKS - _NBUF + b, b).wait()


@jax.jit
def _embed_gather(position_ids_flat, table):
    mesh = plsc.VectorSubcoreMesh(core_axis_name="c", subcore_axis_name="s")
    kern = functools.partial(
        pl.kernel,
        mesh=mesh,
        out_type=jax.ShapeDtypeStruct((_B, EMBED_DIM), jnp.float32),
        scratch_types=[
            pltpu.VMEM((_B_PER_W,), jnp.int32),
            pltpu.VMEM((_CHUNK, EMBED_DIM), jnp.float32),
            pltpu.VMEM((_CHUNK, EMBED_DIM), jnp.float32),
            pltpu.SemaphoreType.DMA,
            pltpu.SemaphoreType.DMA,
            pltpu.SemaphoreType.DMA,
            pltpu.SemaphoreType.DMA,
        ],
    )(_gather_body)
    return kern(position_ids_flat, table)


def kernel(position_ids, table):
    flat = position_ids.reshape(-1).astype(jnp.int32)
    out = _embed_gather(flat, table)
    return out.reshape(BATCH, SEQ_LEN, EMBED_DIM)


# speedup vs baseline: 2.3196x; 1.0266x over previous
import functools

import jax
import jax.numpy as jnp
from jax import lax
from jax.experimental import pallas as pl
from jax.experimental.pallas import tpu as pltpu
from jax.experimental.pallas import tpu_sc as plsc

MAX_POSITION = 8192
EMBED_DIM = 1024
BATCH = 4
SEQ_LEN = 8192

_INFO = plsc.get_sparse_core_info()
_NC = _INFO.num_cores
_NS = _INFO.num_subcores
_NW = _NC * _NS

_B = BATCH * SEQ_LEN
_B_PER_W = _B // _NW
_CHUNK = 16
_NBUF = 4
_N_CHUNKS = _B_PER_W // _CHUNK


def _gather_body(idx_hbm, table_hbm, out_hbm, idx_v, *scratch):
    wid = lax.axis_index("s") * _NC + lax.axis_index("c")
    base = wid * _B_PER_W
    pltpu.sync_copy(idx_hbm.at[pl.ds(base, _B_PER_W)], idx_v)

    rows = scratch[:_NBUF]
    gsem = scratch[_NBUF:2 * _NBUF]
    ssem = scratch[2 * _NBUF:]

    def gather_copy(i, buf):
        off = pl.multiple_of(i * _CHUNK, _CHUNK)
        return pltpu.make_async_copy(
            table_hbm.at[idx_v.at[pl.ds(off, _CHUNK)]], rows[buf], gsem[buf]
        )

    def store_copy(i, buf):
        off = pl.multiple_of(i * _CHUNK, _CHUNK)
        return pltpu.make_async_copy(
            rows[buf], out_hbm.at[pl.ds(base + off, _CHUNK)], ssem[buf]
        )

    for b in range(_NBUF):
        gather_copy(b, b).start()

    def group(g, _):
        cb = g * _NBUF
        for b in range(_NBUF):
            gather_copy(cb + b, b).wait()
            store_copy(cb + b, b).start()
        for b in range(_NBUF):
            @pl.when(cb + _NBUF + b < _N_CHUNKS)
            def _(b=b, cb=cb):
                store_copy(cb + b, b).wait()
                gather_copy(cb + _NBUF + b, b).start()
        return 0

    lax.fori_loop(0, _N_CHUNKS // _NBUF, group, 0)
    for b in range(_NBUF):
        store_copy(_N_CHUNKS - _NBUF + b, b).wait()


@jax.jit
def _embed_gather(position_ids_flat, table):
    mesh = plsc.VectorSubcoreMesh(core_axis_name="c", subcore_axis_name="s")
    kern = functools.partial(
        pl.kernel,
        mesh=mesh,
        out_type=jax.ShapeDtypeStruct((_B, EMBED_DIM), jnp.float32),
        scratch_types=(
            [pltpu.VMEM((_B_PER_W,), jnp.int32)]
            + [pltpu.VMEM((_CHUNK, EMBED_DIM), jnp.float32) for _ in range(_NBUF)]
            + [pltpu.SemaphoreType.DMA for _ in range(2 * _NBUF)]
        ),
    )(_gather_body)
    return kern(position_ids_flat, table)


def kernel(position_ids, table):
    flat = position_ids.reshape(-1).astype(jnp.int32)
    out = _embed_gather(flat, table)
    return out.reshape(BATCH, SEQ_LEN, EMBED_DIM)


# speedup vs baseline: 2.3600x; 1.0174x over previous
import functools

import jax
import jax.numpy as jnp
from jax import lax
from jax.experimental import pallas as pl
from jax.experimental.pallas import tpu as pltpu
from jax.experimental.pallas import tpu_sc as plsc

MAX_POSITION = 8192
EMBED_DIM = 1024
BATCH = 4
SEQ_LEN = 8192

_INFO = plsc.get_sparse_core_info()
_NC = _INFO.num_cores
_NS = _INFO.num_subcores
_NW = _NC * _NS

_B = BATCH * SEQ_LEN
_B_PER_W = _B // _NW
_CHUNK = 8
_NBUF = 8
_N_CHUNKS = _B_PER_W // _CHUNK


def _gather_body(idx_hbm, table_hbm, out_hbm, idx_v, *scratch):
    wid = lax.axis_index("s") * _NC + lax.axis_index("c")
    base = wid * _B_PER_W
    pltpu.sync_copy(idx_hbm.at[pl.ds(base, _B_PER_W)], idx_v)

    rows = scratch[:_NBUF]
    gsem = scratch[_NBUF:2 * _NBUF]
    ssem = scratch[2 * _NBUF:]

    def gather_copy(i, buf):
        off = pl.multiple_of(i * _CHUNK, _CHUNK)
        return pltpu.make_async_copy(
            table_hbm.at[idx_v.at[pl.ds(off, _CHUNK)]], rows[buf], gsem[buf]
        )

    def store_copy(i, buf):
        off = pl.multiple_of(i * _CHUNK, _CHUNK)
        return pltpu.make_async_copy(
            rows[buf], out_hbm.at[pl.ds(base + off, _CHUNK)], ssem[buf]
        )

    for b in range(_NBUF):
        gather_copy(b, b).start()

    def group(g, _):
        cb = g * _NBUF
        for b in range(_NBUF):
            gather_copy(cb + b, b).wait()
            store_copy(cb + b, b).start()
        for b in range(_NBUF):
            @pl.when(cb + _NBUF + b < _N_CHUNKS)
            def _(b=b, cb=cb):
                store_copy(cb + b, b).wait()
                gather_copy(cb + _NBUF + b, b).start()
        return 0

    lax.fori_loop(0, _N_CHUNKS // _NBUF, group, 0)
    for b in range(_NBUF):
        store_copy(_N_CHUNKS - _NBUF + b, b).wait()


@jax.jit
def _embed_gather(position_ids_flat, table):
    mesh = plsc.VectorSubcoreMesh(core_axis_name="c", subcore_axis_name="s")
    kern = functools.partial(
        pl.kernel,
        mesh=mesh,
        out_type=jax.ShapeDtypeStruct((_B, EMBED_DIM), jnp.float32),
        scratch_types=(
            [pltpu.VMEM((_B_PER_W,), jnp.int32)]
            + [pltpu.VMEM((_CHUNK, EMBED_DIM), jnp.float32) for _ in range(_NBUF)]
            + [pltpu.SemaphoreType.DMA for _ in range(2 * _NBUF)]
        ),
    )(_gather_body)
    return kern(position_ids_flat, table)


def kernel(position_ids, table):
    flat = position_ids.reshape(-1).astype(jnp.int32)
    out = _embed_gather(flat, table)
    return out.reshape(BATCH, SEQ_LEN, EMBED_DIM)


# speedup vs baseline: 2.3826x; 1.0096x over previous
import functools

import jax
import jax.numpy as jnp
from jax import lax
from jax.experimental import pallas as pl
from jax.experimental.pallas import tpu as pltpu
from jax.experimental.pallas import tpu_sc as plsc

MAX_POSITION = 8192
EMBED_DIM = 1024
BATCH = 4
SEQ_LEN = 8192

_INFO = plsc.get_sparse_core_info()
_NC = _INFO.num_cores
_NS = _INFO.num_subcores
_NW = _NC * _NS

_B = BATCH * SEQ_LEN
_B_PER_W = _B // _NW
_CHUNK = 8
_NBUF = 8
_LAG = _NBUF // 2
_N_CHUNKS = _B_PER_W // _CHUNK


def _gather_body(idx_hbm, table_hbm, out_hbm, idx_v, *scratch):
    wid = lax.axis_index("s") * _NC + lax.axis_index("c")
    base = wid * _B_PER_W
    pltpu.sync_copy(idx_hbm.at[pl.ds(base, _B_PER_W)], idx_v)

    rows = scratch[:_NBUF]
    gsem = scratch[_NBUF:2 * _NBUF]
    ssem = scratch[2 * _NBUF:]

    def gather_copy(i, buf):
        off = pl.multiple_of(i * _CHUNK, _CHUNK)
        return pltpu.make_async_copy(
            table_hbm.at[idx_v.at[pl.ds(off, _CHUNK)]], rows[buf], gsem[buf]
        )

    def store_copy(i, buf):
        off = pl.multiple_of(i * _CHUNK, _CHUNK)
        return pltpu.make_async_copy(
            rows[buf], out_hbm.at[pl.ds(base + off, _CHUNK)], ssem[buf]
        )

    def a_step(i, b):
        gather_copy(i, b).wait()
        store_copy(i, b).start()

    def b_step(i, b):
        store_copy(i, b).wait()
        gather_copy(i + _NBUF, b).start()

    for b in range(_NBUF):
        gather_copy(b, b).start()

    for b in range(_NBUF):
        a_step(b, b)
        if b >= _LAG:
            b_step(b - _LAG, b - _LAG)

    def group(g, _):
        cb = g * _NBUF
        for b in range(_NBUF):
            a_step(cb + b, b)
            k = cb + b - _LAG

            @pl.when(k + _NBUF < _N_CHUNKS)
            def _(k=k, b=b):
                b_step(k, (b - _LAG) % _NBUF)
        return 0

    lax.fori_loop(1, _N_CHUNKS // _NBUF, group, 0)
    for b in range(_NBUF):
        store_copy(_N_CHUNKS - _NBUF + b, b).wait()


@jax.jit
def _embed_gather(position_ids_flat, table):
    mesh = plsc.VectorSubcoreMesh(core_axis_name="c", subcore_axis_name="s")
    kern = functools.partial(
        pl.kernel,
        mesh=mesh,
        out_type=jax.ShapeDtypeStruct((_B, EMBED_DIM), jnp.float32),
        scratch_types=(
            [pltpu.VMEM((_B_PER_W,), jnp.int32)]
            + [pltpu.VMEM((_CHUNK, EMBED_DIM), jnp.float32) for _ in range(_NBUF)]
            + [pltpu.SemaphoreType.DMA for _ in range(2 * _NBUF)]
        ),
    )(_gather_body)
    return kern(position_ids_flat, table)


def kernel(position_ids, table):
    flat = position_ids.reshape(-1).astype(jnp.int32)
    out = _embed_gather(flat, table)
    return out.reshape(BATCH, SEQ_LEN, EMBED_DIM)


# speedup vs baseline: 2.3866x; 1.0017x over previous
import functools

import jax
import jax.numpy as jnp
from jax import lax
from jax.experimental import pallas as pl
from jax.experimental.pallas import tpu as pltpu
from jax.experimental.pallas import tpu_sc as plsc

MAX_POSITION = 8192
EMBED_DIM = 1024
BATCH = 4
SEQ_LEN = 8192

_INFO = plsc.get_sparse_core_info()
_NC = _INFO.num_cores
_NS = _INFO.num_subcores
_NW = _NC * _NS

_B = BATCH * SEQ_LEN
_B_PER_W = _B // _NW
_CHUNK = 8
_NBUF = 8
_LAG = 2
_N_CHUNKS = _B_PER_W // _CHUNK


def _gather_body(idx_hbm, table_hbm, out_hbm, idx_v, *scratch):
    wid = lax.axis_index("s") * _NC + lax.axis_index("c")
    base = wid * _B_PER_W
    pltpu.sync_copy(idx_hbm.at[pl.ds(base, _B_PER_W)], idx_v)

    rows = scratch[:_NBUF]
    gsem = scratch[_NBUF:2 * _NBUF]
    ssem = scratch[2 * _NBUF:]

    def gather_copy(i, buf):
        off = pl.multiple_of(i * _CHUNK, _CHUNK)
        return pltpu.make_async_copy(
            table_hbm.at[idx_v.at[pl.ds(off, _CHUNK)]], rows[buf], gsem[buf]
        )

    def store_copy(i, buf):
        off = pl.multiple_of(i * _CHUNK, _CHUNK)
        return pltpu.make_async_copy(
            rows[buf], out_hbm.at[pl.ds(base + off, _CHUNK)], ssem[buf]
        )

    def a_step(i, b):
        gather_copy(i, b).wait()
        store_copy(i, b).start()

    def b_step(i, b):
        store_copy(i, b).wait()
        gather_copy(i + _NBUF, b).start()

    for b in range(_NBUF):
        gather_copy(b, b).start()

    for b in range(_NBUF):
        a_step(b, b)
        if b >= _LAG:
            b_step(b - _LAG, b - _LAG)

    def group(g, _):
        cb = g * _NBUF
        for b in range(_NBUF):
            a_step(cb + b, b)
            k = cb + b - _LAG

            @pl.when(k + _NBUF < _N_CHUNKS)
            def _(k=k, b=b):
                b_step(k, (b - _LAG) % _NBUF)
        return 0

    lax.fori_loop(1, _N_CHUNKS // _NBUF, group, 0)
    for b in range(_NBUF):
        store_copy(_N_CHUNKS - _NBUF + b, b).wait()


@jax.jit
def _embed_gather(position_ids_flat, table):
    mesh = plsc.VectorSubcoreMesh(core_axis_name="c", subcore_axis_name="s")
    kern = functools.partial(
        pl.kernel,
        mesh=mesh,
        out_type=jax.ShapeDtypeStruct((_B, EMBED_DIM), jnp.float32),
        scratch_types=(
            [pltpu.VMEM((_B_PER_W,), jnp.int32)]
            + [pltpu.VMEM((_CHUNK, EMBED_DIM), jnp.float32) for _ in range(_NBUF)]
            + [pltpu.SemaphoreType.DMA for _ in range(2 * _NBUF)]
        ),
    )(_gather_body)
    return kern(position_ids_flat, table)


def kernel(position_ids, table):
    flat = position_ids.reshape(-1).astype(jnp.int32)
    out = _embed_gather(flat, table)
    return out.reshape(BATCH, SEQ_LEN, EMBED_DIM)
